# Initial kernel scaffold; baseline (speedup 1.0000x reference)
import jax
import jax.numpy as jnp
from jax.experimental import pallas as pl


def kernel(x, table, dividing_points):
    raise NotImplementedError("write your pallas kernel here")


# speedup vs baseline: 972.8996x; 972.8996x over previous
import functools

import jax
import jax.numpy as jnp
from jax import lax
from jax.experimental import pallas as pl
from jax.experimental.pallas import tpu as pltpu
from jax.experimental.pallas import tpu_sc as plsc

NCORES = 2
NSUB = 16
NWORK = NCORES * NSUB
LANES = 16
SEGS = 6
TLEN = 64
CH = 16384


def _sc_lut(x, tab_flat, consts):
    n = x.shape[0]
    per_w = n // NWORK
    n_chunks = per_w // CH

    mesh = plsc.VectorSubcoreMesh(
        core_axis_name="c", subcore_axis_name="s",
        num_cores=NCORES, num_subcores=NSUB)

    @functools.partial(
        pl.kernel,
        out_type=jax.ShapeDtypeStruct((n,), jnp.float32),
        mesh=mesh,
        scratch_types=[
            pltpu.VMEM((SEGS * TLEN,), jnp.float32),
            pltpu.VMEM((3, LANES), jnp.float32),
            pltpu.VMEM((CH,), jnp.float32),
            pltpu.VMEM((CH,), jnp.float32),
        ],
        compiler_params=pltpu.CompilerParams(needs_layout_passes=False),
    )
    def k(x_hbm, tab_hbm, consts_hbm, out_hbm, tab_v, c_v, in_v, out_v):
        wid = lax.axis_index("s") * NCORES + lax.axis_index("c")
        base = wid * per_w
        pltpu.sync_copy(tab_hbm, tab_v)
        pltpu.sync_copy(consts_hbm, c_v)
        lo0 = c_v[0]
        hi0 = c_v[1]
        invw = c_v[2]

        def chunk_body(g, _):
            start = base + g * CH
            pltpu.sync_copy(x_hbm.at[pl.ds(start, CH)], in_v)

            def vec_body(i, _):
                xv = in_v[pl.ds(i * LANES, LANES)]
                xc = jnp.minimum(jnp.maximum(xv, lo0), hi0)
                u = (xc - lo0) * invw
                segi = jnp.minimum(u.astype(jnp.int32), SEGS - 1)
                pos = (u - segi.astype(jnp.float32)) * float(TLEN - 1)
                idx0 = jnp.minimum(pos.astype(jnp.int32), TLEN - 2)
                frac = pos - idx0.astype(jnp.float32)
                basei = segi * TLEN + idx0
                y0 = plsc.load_gather(tab_v, [basei])
                y1 = plsc.load_gather(tab_v, [basei + 1])
                out_v[pl.ds(i * LANES, LANES)] = y0 * (1.0 - frac) + y1 * frac
                return 0

            lax.fori_loop(0, CH // LANES, vec_body, 0)
            pltpu.sync_copy(out_v, out_hbm.at[pl.ds(start, CH)])
            return 0

        lax.fori_loop(0, n_chunks, chunk_body, 0)

    return k(x, tab_flat, consts)


def kernel(x, table, dividing_points):
    tab_flat = table.reshape(-1)
    lo0 = dividing_points[0]
    hi0 = dividing_points[-1]
    invw = SEGS / (hi0 - lo0)
    consts = jnp.stack([
        jnp.full((LANES,), lo0, jnp.float32),
        jnp.full((LANES,), hi0, jnp.float32),
        jnp.full((LANES,), invw, jnp.float32),
    ])
    return _sc_lut(x, tab_flat, consts)
